# baseline (speedup 1.0000x reference)
import functools

import jax
import jax.numpy as jnp
from jax import lax
from jax.experimental import pallas as pl
from jax.experimental.pallas import tpu as pltpu
import jax.experimental.pallas.tpu_sc as plsc

T = 4096
D = 1024
F = 4096
E = 8
CAP = 640
SLOT = 656
NROWS = E * SLOT
FB = 512
NF = F // FB
CHUNK = 512
NCHUNK = T // CHUNK

NC = 2
NS = 16
NW = NC * NS
TPW = T // NW
SUB = 64
NSUB = TPW // SUB


def _router_body(x_ref, wg_ref, xs_ref, idx_ref, carry_ref):
    c = pl.program_id(0)

    @pl.when(c == 0)
    def _():
        carry_ref[...] = jnp.zeros((1, E), dtype=jnp.float32)

    x = x_ref[...]
    wg = wg_ref[...]
    logits = jnp.dot(x.astype(jnp.bfloat16), wg.astype(jnp.bfloat16),
                     preferred_element_type=jnp.float32)
    m = jnp.max(logits, axis=1, keepdims=True)
    p = jnp.exp(logits - m)
    probs = p / jnp.sum(p, axis=1, keepdims=True)
    gate = jnp.max(probs, axis=1, keepdims=True)
    e_iota = lax.broadcasted_iota(jnp.int32, (CHUNK, E), 1)
    is_max = probs >= gate
    eidx = jnp.min(jnp.where(is_max, e_iota, E), axis=1, keepdims=True)
    mask = (e_iota == eidx).astype(jnp.float32)

    ci = lax.broadcasted_iota(jnp.int32, (CHUNK, CHUNK), 0)
    cj = lax.broadcasted_iota(jnp.int32, (CHUNK, CHUNK), 1)
    tri = (ci >= cj).astype(jnp.float32)
    cum = jnp.dot(tri, mask, preferred_element_type=jnp.float32) + carry_ref[...]
    carry_ref[...] = cum[CHUNK - 1:CHUNK, :]

    pos = jnp.sum(cum * mask, axis=1, keepdims=True) - 1.0
    pos_i = pos.astype(jnp.int32)
    keep = (pos_i < CAP)
    slot = jnp.where(keep, pos_i, CAP)
    idx_ref[...] = eidx * SLOT + slot
    gk = jnp.where(keep, gate, 0.0)
    xs_ref[...] = (x * gk).astype(jnp.bfloat16)


def _router(x, wg):
    return pl.pallas_call(
        _router_body,
        grid=(NCHUNK,),
        in_specs=[
            pl.BlockSpec((CHUNK, D), lambda c: (c, 0)),
            pl.BlockSpec((D, E), lambda c: (0, 0)),
        ],
        out_specs=(pl.BlockSpec((CHUNK, D), lambda c: (c, 0)),
                   pl.BlockSpec((CHUNK, 1), lambda c: (c, 0))),
        out_shape=(jax.ShapeDtypeStruct((T, D), jnp.bfloat16),
                   jax.ShapeDtypeStruct((T, 1), jnp.int32)),
        scratch_shapes=[pltpu.VMEM((1, E), jnp.float32)],
    )(x, wg)


def _ffn_body(disp_ref, wi_ref, wo_ref, eo_ref):
    fb = pl.program_id(1)
    xb = disp_ref[...]
    wi = wi_ref[0].astype(jnp.bfloat16)
    wo = wo_ref[0].astype(jnp.bfloat16)
    h = jnp.maximum(
        jnp.dot(xb, wi, preferred_element_type=jnp.float32), 0.0)
    contrib = jnp.dot(h.astype(jnp.bfloat16), wo,
                      preferred_element_type=jnp.float32)

    @pl.when(fb == 0)
    def _():
        eo_ref[...] = contrib

    @pl.when(fb != 0)
    def _():
        eo_ref[...] += contrib


def _ffn(disp, wi, wo):
    return pl.pallas_call(
        _ffn_body,
        grid=(E, NF),
        in_specs=[
            pl.BlockSpec((SLOT, D), lambda e, f: (e, 0)),
            pl.BlockSpec((1, D, FB), lambda e, f: (e, 0, f)),
            pl.BlockSpec((1, FB, D), lambda e, f: (e, f, 0)),
        ],
        out_specs=pl.BlockSpec((SLOT, D), lambda e, f: (e, 0)),
        out_shape=jax.ShapeDtypeStruct((NROWS, D), jnp.float32),
    )(disp, wi, wo)


@functools.cache
def _sc_kernels():
    mesh = plsc.VectorSubcoreMesh(core_axis_name="c", subcore_axis_name="s",
                                  num_cores=NC, num_subcores=NS)

    @functools.partial(
        pl.kernel,
        out_type=jax.ShapeDtypeStruct((NROWS, D // 2), jnp.int32),
        mesh=mesh,
        scratch_types=[
            pltpu.VMEM((1, TPW), jnp.int32),
            pltpu.VMEM((TPW, D // 2), jnp.int32),
            pltpu.SemaphoreType.DMA,
        ])
    def scatter_k(xs_hbm, idx_hbm, disp_hbm, idx_v, rows_v, sem):
        wid = lax.axis_index("s") * NC + lax.axis_index("c")
        base = wid * TPW
        pltpu.sync_copy(idx_hbm.at[pl.ds(base, TPW)], idx_v.at[0])
        pltpu.sync_copy(xs_hbm.at[pl.ds(base, TPW)], rows_v)
        pltpu.async_copy(rows_v, disp_hbm.at[idx_v.at[0]], sem).wait()

    @functools.partial(
        pl.kernel,
        out_type=jax.ShapeDtypeStruct((T, D), jnp.float32),
        mesh=mesh,
        scratch_types=[
            pltpu.VMEM((NSUB, SUB), jnp.int32),
            pltpu.VMEM((SUB, D), jnp.float32),
            pltpu.SemaphoreType.DMA,
        ])
    def gather_k(eo_hbm, idx_hbm, out_hbm, idx_v, rows_v, sem):
        wid = lax.axis_index("s") * NC + lax.axis_index("c")
        base = wid * TPW
        for j in range(NSUB):
            pltpu.sync_copy(idx_hbm.at[pl.ds(base + j * SUB, SUB)], idx_v.at[j])
            pltpu.async_copy(eo_hbm.at[idx_v.at[j]], rows_v, sem).wait()
            pltpu.sync_copy(rows_v, out_hbm.at[pl.ds(base + j * SUB, SUB)])

    return scatter_k, gather_k


def _sc_scatter(xs, idx):
    return _sc_kernels()[0](xs, idx)


def _sc_gather(eo, idx):
    return _sc_kernels()[1](eo, idx)


def kernel(x, Wg, wi, wo):
    xs, idx2d = _router(x, Wg)
    idx = idx2d.reshape(T)
    xs_bits = lax.bitcast_convert_type(xs.reshape(T, D // 2, 2), jnp.int32)
    disp_bits = _sc_scatter(xs_bits, idx)
    disp = lax.bitcast_convert_type(disp_bits, jnp.bfloat16).reshape(NROWS, D)
    eo = _ffn(disp, wi, wo)
    return _sc_gather(eo, idx)


# speedup vs baseline: 2.3185x; 2.3185x over previous
import functools

import jax
import jax.numpy as jnp
from jax import lax
from jax.experimental import pallas as pl
from jax.experimental.pallas import tpu as pltpu
import jax.experimental.pallas.tpu_sc as plsc

T = 4096
D = 1024
F = 4096
E = 8
CAP = 640
SLOT = 648
NROWS = E * SLOT
FB = 2048
NF = F // FB
CHUNK = 512
NCHUNK = T // CHUNK

NC = 2
NS = 16
NW = NC * NS
TPW = T // NW
SUB = 64
NSUB = TPW // SUB


def _router_body(x_ref, wg_ref, xs_ref, idx_ref, carry_ref):
    c = pl.program_id(0)

    @pl.when(c == 0)
    def _():
        carry_ref[...] = jnp.zeros((1, E), dtype=jnp.float32)

    x = x_ref[...]
    wg = wg_ref[...]
    logits = jnp.dot(x.astype(jnp.bfloat16), wg.astype(jnp.bfloat16),
                     preferred_element_type=jnp.float32)
    m = jnp.max(logits, axis=1, keepdims=True)
    p = jnp.exp(logits - m)
    probs = p / jnp.sum(p, axis=1, keepdims=True)
    gate = jnp.max(probs, axis=1, keepdims=True)
    e_iota = lax.broadcasted_iota(jnp.int32, (CHUNK, E), 1)
    is_max = probs >= gate
    eidx = jnp.min(jnp.where(is_max, e_iota, E), axis=1, keepdims=True)
    mask = (e_iota == eidx).astype(jnp.float32)

    ci = lax.broadcasted_iota(jnp.int32, (CHUNK, CHUNK), 0)
    cj = lax.broadcasted_iota(jnp.int32, (CHUNK, CHUNK), 1)
    tri = (ci >= cj).astype(jnp.float32)
    cum = jnp.dot(tri, mask, preferred_element_type=jnp.float32) + carry_ref[...]
    carry_ref[...] = cum[CHUNK - 1:CHUNK, :]

    pos = jnp.sum(cum * mask, axis=1, keepdims=True) - 1.0
    pos_i = pos.astype(jnp.int32)
    keep = (pos_i < CAP)
    slot = jnp.where(keep, pos_i, CAP)
    idx_ref[...] = eidx * SLOT + slot
    gk = jnp.where(keep, gate, 0.0)
    xs_ref[...] = x * gk


def _router(x, wg):
    return pl.pallas_call(
        _router_body,
        grid=(NCHUNK,),
        in_specs=[
            pl.BlockSpec((CHUNK, D), lambda c: (c, 0)),
            pl.BlockSpec((D, E), lambda c: (0, 0)),
        ],
        out_specs=(pl.BlockSpec((CHUNK, D), lambda c: (c, 0)),
                   pl.BlockSpec((CHUNK, 1), lambda c: (c, 0))),
        out_shape=(jax.ShapeDtypeStruct((T, D), jnp.float32),
                   jax.ShapeDtypeStruct((T, 1), jnp.int32)),
        scratch_shapes=[pltpu.VMEM((1, E), jnp.float32)],
    )(x, wg)


def _ffn_body(disp_ref, wi_ref, wo_ref, eo_ref):
    fb = pl.program_id(1)
    xb = disp_ref[...].astype(jnp.bfloat16)
    wi = wi_ref[0].astype(jnp.bfloat16)
    wo = wo_ref[0].astype(jnp.bfloat16)
    h = jnp.maximum(
        jnp.dot(xb, wi, preferred_element_type=jnp.float32), 0.0)
    contrib = jnp.dot(h.astype(jnp.bfloat16), wo,
                      preferred_element_type=jnp.float32)

    @pl.when(fb == 0)
    def _():
        eo_ref[...] = contrib

    @pl.when(fb != 0)
    def _():
        eo_ref[...] += contrib


def _ffn(disp, wi, wo):
    return pl.pallas_call(
        _ffn_body,
        grid=(E, NF),
        in_specs=[
            pl.BlockSpec((SLOT, D), lambda e, f: (e, 0)),
            pl.BlockSpec((1, D, FB), lambda e, f: (e, 0, f)),
            pl.BlockSpec((1, FB, D), lambda e, f: (e, f, 0)),
        ],
        out_specs=pl.BlockSpec((SLOT, D), lambda e, f: (e, 0)),
        out_shape=jax.ShapeDtypeStruct((NROWS, D), jnp.float32),
    )(disp, wi, wo)


@functools.cache
def _sc_kernels():
    mesh = plsc.VectorSubcoreMesh(core_axis_name="c", subcore_axis_name="s",
                                  num_cores=NC, num_subcores=NS)

    @functools.partial(
        pl.kernel,
        out_type=jax.ShapeDtypeStruct((NROWS, D), jnp.float32),
        mesh=mesh,
        scratch_types=[
            pltpu.VMEM((NSUB, SUB), jnp.int32),
            pltpu.VMEM((SUB, D), jnp.float32),
            pltpu.SemaphoreType.DMA,
        ])
    def scatter_k(xs_hbm, idx_hbm, disp_hbm, idx_v, rows_v, sem):
        wid = lax.axis_index("s") * NC + lax.axis_index("c")
        base = wid * TPW
        for j in range(NSUB):
            pltpu.sync_copy(idx_hbm.at[pl.ds(base + j * SUB, SUB)], idx_v.at[j])
            pltpu.sync_copy(xs_hbm.at[pl.ds(base + j * SUB, SUB)], rows_v)
            pltpu.async_copy(rows_v, disp_hbm.at[idx_v.at[j]], sem).wait()

    @functools.partial(
        pl.kernel,
        out_type=jax.ShapeDtypeStruct((T, D), jnp.float32),
        mesh=mesh,
        scratch_types=[
            pltpu.VMEM((NSUB, SUB), jnp.int32),
            pltpu.VMEM((SUB, D), jnp.float32),
            pltpu.SemaphoreType.DMA,
        ])
    def gather_k(eo_hbm, idx_hbm, out_hbm, idx_v, rows_v, sem):
        wid = lax.axis_index("s") * NC + lax.axis_index("c")
        base = wid * TPW
        for j in range(NSUB):
            pltpu.sync_copy(idx_hbm.at[pl.ds(base + j * SUB, SUB)], idx_v.at[j])
            pltpu.async_copy(eo_hbm.at[idx_v.at[j]], rows_v, sem).wait()
            pltpu.sync_copy(rows_v, out_hbm.at[pl.ds(base + j * SUB, SUB)])

    return scatter_k, gather_k


def _sc_scatter(xs, idx):
    return _sc_kernels()[0](xs, idx)


def _sc_gather(eo, idx):
    return _sc_kernels()[1](eo, idx)


def kernel(x, Wg, wi, wo):
    xs, idx2d = _router(x, Wg)
    idx = idx2d.reshape(T)
    disp = _sc_scatter(xs, idx)
    eo = _ffn(disp, wi, wo)
    return _sc_gather(eo, idx)


# speedup vs baseline: 2.3230x; 1.0019x over previous
import functools

import jax
import jax.numpy as jnp
from jax import lax
from jax.experimental import pallas as pl
from jax.experimental.pallas import tpu as pltpu
import jax.experimental.pallas.tpu_sc as plsc

T = 4096
D = 1024
F = 4096
E = 8
CAP = 640
SLOT = 648
NROWS = E * SLOT
FB = 2048
NF = F // FB
CHUNK = 512
NCHUNK = T // CHUNK

NC = 2
NS = 16
NW = NC * NS
TPW = T // NW
SUB = 64
NSUB = TPW // SUB


def _router_body(x_ref, wg_ref, xs_ref, idx_ref, carry_ref):
    c = pl.program_id(0)

    @pl.when(c == 0)
    def _():
        carry_ref[...] = jnp.zeros((1, E), dtype=jnp.float32)

    x = x_ref[...]
    wg = wg_ref[...]
    logits = jnp.dot(x.astype(jnp.bfloat16), wg.astype(jnp.bfloat16),
                     preferred_element_type=jnp.float32)
    m = jnp.max(logits, axis=1, keepdims=True)
    p = jnp.exp(logits - m)
    probs = p / jnp.sum(p, axis=1, keepdims=True)
    gate = jnp.max(probs, axis=1, keepdims=True)
    e_iota = lax.broadcasted_iota(jnp.int32, (CHUNK, E), 1)
    is_max = probs >= gate
    eidx = jnp.min(jnp.where(is_max, e_iota, E), axis=1, keepdims=True)
    mask = (e_iota == eidx).astype(jnp.float32)

    ci = lax.broadcasted_iota(jnp.int32, (CHUNK, CHUNK), 0)
    cj = lax.broadcasted_iota(jnp.int32, (CHUNK, CHUNK), 1)
    tri = (ci >= cj).astype(jnp.float32)
    cum = jnp.dot(tri, mask, preferred_element_type=jnp.float32) + carry_ref[...]
    carry_ref[...] = cum[CHUNK - 1:CHUNK, :]

    pos = jnp.sum(cum * mask, axis=1, keepdims=True) - 1.0
    pos_i = pos.astype(jnp.int32)
    keep = (pos_i < CAP)
    slot = jnp.where(keep, pos_i, CAP)
    idx_ref[...] = eidx * SLOT + slot
    gk = jnp.where(keep, gate, 0.0)
    xs_ref[...] = x * gk


def _router(x, wg):
    return pl.pallas_call(
        _router_body,
        grid=(NCHUNK,),
        in_specs=[
            pl.BlockSpec((CHUNK, D), lambda c: (c, 0)),
            pl.BlockSpec((D, E), lambda c: (0, 0)),
        ],
        out_specs=(pl.BlockSpec((CHUNK, D), lambda c: (c, 0)),
                   pl.BlockSpec((CHUNK, 1), lambda c: (c, 0))),
        out_shape=(jax.ShapeDtypeStruct((T, D), jnp.float32),
                   jax.ShapeDtypeStruct((T, 1), jnp.int32)),
        scratch_shapes=[pltpu.VMEM((1, E), jnp.float32)],
    )(x, wg)


def _ffn_body(disp_ref, wi_ref, wo_ref, eo_ref):
    fb = pl.program_id(1)
    xb = disp_ref[...]
    wi = wi_ref[0]
    wo = wo_ref[0]
    h = jnp.maximum(
        jnp.dot(xb, wi, preferred_element_type=jnp.float32,
                precision=lax.Precision.DEFAULT), 0.0)
    contrib = jnp.dot(h, wo, preferred_element_type=jnp.float32,
                      precision=lax.Precision.DEFAULT)

    @pl.when(fb == 0)
    def _():
        eo_ref[...] = contrib

    @pl.when(fb != 0)
    def _():
        eo_ref[...] += contrib


def _ffn(disp, wi, wo):
    return pl.pallas_call(
        _ffn_body,
        grid=(E, NF),
        in_specs=[
            pl.BlockSpec((SLOT, D), lambda e, f: (e, 0)),
            pl.BlockSpec((1, D, FB), lambda e, f: (e, 0, f)),
            pl.BlockSpec((1, FB, D), lambda e, f: (e, f, 0)),
        ],
        out_specs=pl.BlockSpec((SLOT, D), lambda e, f: (e, 0)),
        out_shape=jax.ShapeDtypeStruct((NROWS, D), jnp.float32),
    )(disp, wi, wo)


@functools.cache
def _sc_kernels():
    mesh = plsc.VectorSubcoreMesh(core_axis_name="c", subcore_axis_name="s",
                                  num_cores=NC, num_subcores=NS)

    @functools.partial(
        pl.kernel,
        out_type=jax.ShapeDtypeStruct((NROWS, D), jnp.float32),
        mesh=mesh,
        scratch_types=[
            pltpu.VMEM((NSUB, SUB), jnp.int32),
            pltpu.VMEM((SUB, D), jnp.float32),
            pltpu.SemaphoreType.DMA,
        ])
    def scatter_k(xs_hbm, idx_hbm, disp_hbm, idx_v, rows_v, sem):
        wid = lax.axis_index("s") * NC + lax.axis_index("c")
        base = wid * TPW
        for j in range(NSUB):
            pltpu.sync_copy(idx_hbm.at[pl.ds(base + j * SUB, SUB)], idx_v.at[j])
            pltpu.sync_copy(xs_hbm.at[pl.ds(base + j * SUB, SUB)], rows_v)
            pltpu.async_copy(rows_v, disp_hbm.at[idx_v.at[j]], sem).wait()

    @functools.partial(
        pl.kernel,
        out_type=jax.ShapeDtypeStruct((T, D), jnp.float32),
        mesh=mesh,
        scratch_types=[
            pltpu.VMEM((NSUB, SUB), jnp.int32),
            pltpu.VMEM((SUB, D), jnp.float32),
            pltpu.SemaphoreType.DMA,
        ])
    def gather_k(eo_hbm, idx_hbm, out_hbm, idx_v, rows_v, sem):
        wid = lax.axis_index("s") * NC + lax.axis_index("c")
        base = wid * TPW
        for j in range(NSUB):
            pltpu.sync_copy(idx_hbm.at[pl.ds(base + j * SUB, SUB)], idx_v.at[j])
            pltpu.async_copy(eo_hbm.at[idx_v.at[j]], rows_v, sem).wait()
            pltpu.sync_copy(rows_v, out_hbm.at[pl.ds(base + j * SUB, SUB)])

    return scatter_k, gather_k


def _sc_scatter(xs, idx):
    return _sc_kernels()[0](xs, idx)


def _sc_gather(eo, idx):
    return _sc_kernels()[1](eo, idx)


def kernel(x, Wg, wi, wo):
    xs, idx2d = _router(x, Wg)
    idx = idx2d.reshape(T)
    disp = _sc_scatter(xs, idx)
    eo = _ffn(disp, wi, wo)
    return _sc_gather(eo, idx)


# speedup vs baseline: 2.4029x; 1.0344x over previous
import functools

import jax
import jax.numpy as jnp
from jax import lax
from jax.experimental import pallas as pl
from jax.experimental.pallas import tpu as pltpu
import jax.experimental.pallas.tpu_sc as plsc

T = 4096
D = 1024
D2 = D // 2
F = 4096
E = 8
CAP = 640
SLOT = 648
NROWS = E * SLOT
FB = 2048
NF = F // FB
CHUNK = 512
NCHUNK = T // CHUNK

NC = 2
NS = 16
NW = NC * NS
TPW = T // NW
SUB = 64
NSUB = TPW // SUB


def _router_body(x_ref, wg_ref, xs_ref, idx_ref, carry_ref):
    c = pl.program_id(0)

    @pl.when(c == 0)
    def _():
        carry_ref[...] = jnp.zeros((1, E), dtype=jnp.float32)

    x = x_ref[...]
    wg = wg_ref[...]
    logits = jnp.dot(x.astype(jnp.bfloat16), wg.astype(jnp.bfloat16),
                     preferred_element_type=jnp.float32)
    m = jnp.max(logits, axis=1, keepdims=True)
    p = jnp.exp(logits - m)
    probs = p / jnp.sum(p, axis=1, keepdims=True)
    gate = jnp.max(probs, axis=1, keepdims=True)
    e_iota = lax.broadcasted_iota(jnp.int32, (CHUNK, E), 1)
    is_max = probs >= gate
    eidx = jnp.min(jnp.where(is_max, e_iota, E), axis=1, keepdims=True)
    mask = (e_iota == eidx).astype(jnp.float32)

    ci = lax.broadcasted_iota(jnp.int32, (CHUNK, CHUNK), 0)
    cj = lax.broadcasted_iota(jnp.int32, (CHUNK, CHUNK), 1)
    tri = (ci >= cj).astype(jnp.float32)
    cum = jnp.dot(tri, mask, preferred_element_type=jnp.float32) + carry_ref[...]
    carry_ref[...] = cum[CHUNK - 1:CHUNK, :]

    pos = jnp.sum(cum * mask, axis=1, keepdims=True) - 1.0
    pos_i = pos.astype(jnp.int32)
    keep = (pos_i < CAP)
    slot = jnp.where(keep, pos_i, CAP)
    idx_ref[...] = eidx * SLOT + slot
    gk = jnp.where(keep, gate, 0.0)
    xs_r = (x * gk).astype(jnp.bfloat16).astype(jnp.float32)
    b = lax.bitcast_convert_type(xs_r, jnp.uint32)
    packed = (b[:, :D2] >> 16) | (b[:, D2:] & jnp.uint32(0xFFFF0000))
    xs_ref[...] = packed


def _router(x, wg):
    return pl.pallas_call(
        _router_body,
        grid=(NCHUNK,),
        in_specs=[
            pl.BlockSpec((CHUNK, D), lambda c: (c, 0)),
            pl.BlockSpec((D, E), lambda c: (0, 0)),
        ],
        out_specs=(pl.BlockSpec((CHUNK, D2), lambda c: (c, 0)),
                   pl.BlockSpec((CHUNK, 1), lambda c: (c, 0))),
        out_shape=(jax.ShapeDtypeStruct((T, D2), jnp.uint32),
                   jax.ShapeDtypeStruct((T, 1), jnp.int32)),
        scratch_shapes=[pltpu.VMEM((1, E), jnp.float32)],
    )(x, wg)


def _ffn_body(disp_ref, wi_ref, wo_ref, eo_ref):
    fb = pl.program_id(1)
    u = disp_ref[...]
    x_lo = lax.bitcast_convert_type(u << 16, jnp.float32)
    x_hi = lax.bitcast_convert_type(u & jnp.uint32(0xFFFF0000),
                                    jnp.float32)
    wi = wi_ref[0]
    wo = wo_ref[0]
    acc = jnp.dot(x_lo, wi[:D2], preferred_element_type=jnp.float32,
                  precision=lax.Precision.DEFAULT)
    acc += jnp.dot(x_hi, wi[D2:], preferred_element_type=jnp.float32,
                   precision=lax.Precision.DEFAULT)
    h = jnp.maximum(acc, 0.0)
    contrib = jnp.dot(h, wo, preferred_element_type=jnp.float32,
                      precision=lax.Precision.DEFAULT)

    @pl.when(fb == 0)
    def _():
        eo_ref[...] = contrib

    @pl.when(fb != 0)
    def _():
        eo_ref[...] += contrib


def _ffn(disp, wi, wo):
    return pl.pallas_call(
        _ffn_body,
        grid=(E, NF),
        in_specs=[
            pl.BlockSpec((SLOT, D2), lambda e, f: (e, 0)),
            pl.BlockSpec((1, D, FB), lambda e, f: (e, 0, f)),
            pl.BlockSpec((1, FB, D), lambda e, f: (e, f, 0)),
        ],
        out_specs=pl.BlockSpec((SLOT, D), lambda e, f: (e, 0)),
        out_shape=jax.ShapeDtypeStruct((NROWS, D), jnp.float32),
    )(disp, wi, wo)


@functools.cache
def _sc_kernels():
    mesh = plsc.VectorSubcoreMesh(core_axis_name="c", subcore_axis_name="s",
                                  num_cores=NC, num_subcores=NS)

    @functools.partial(
        pl.kernel,
        out_type=jax.ShapeDtypeStruct((NROWS, D2), jnp.uint32),
        mesh=mesh,
        scratch_types=[
            pltpu.VMEM((NSUB, SUB), jnp.int32),
            pltpu.VMEM((SUB, D2), jnp.uint32),
            pltpu.SemaphoreType.DMA,
        ])
    def scatter_k(xs_hbm, idx_hbm, disp_hbm, idx_v, rows_v, sem):
        wid = lax.axis_index("s") * NC + lax.axis_index("c")
        base = wid * TPW
        for j in range(NSUB):
            pltpu.sync_copy(idx_hbm.at[pl.ds(base + j * SUB, SUB)], idx_v.at[j])
            pltpu.sync_copy(xs_hbm.at[pl.ds(base + j * SUB, SUB)], rows_v)
            pltpu.async_copy(rows_v, disp_hbm.at[idx_v.at[j]], sem).wait()

    @functools.partial(
        pl.kernel,
        out_type=jax.ShapeDtypeStruct((T, D), jnp.float32),
        mesh=mesh,
        scratch_types=[
            pltpu.VMEM((NSUB, SUB), jnp.int32),
            pltpu.VMEM((SUB, D), jnp.float32),
            pltpu.SemaphoreType.DMA,
        ])
    def gather_k(eo_hbm, idx_hbm, out_hbm, idx_v, rows_v, sem):
        wid = lax.axis_index("s") * NC + lax.axis_index("c")
        base = wid * TPW
        for j in range(NSUB):
            pltpu.sync_copy(idx_hbm.at[pl.ds(base + j * SUB, SUB)], idx_v.at[j])
            pltpu.async_copy(eo_hbm.at[idx_v.at[j]], rows_v, sem).wait()
            pltpu.sync_copy(rows_v, out_hbm.at[pl.ds(base + j * SUB, SUB)])

    return scatter_k, gather_k


def _sc_scatter(xs, idx):
    return _sc_kernels()[0](xs, idx)


def _sc_gather(eo, idx):
    return _sc_kernels()[1](eo, idx)


def kernel(x, Wg, wi, wo):
    xs, idx2d = _router(x, Wg)
    idx = idx2d.reshape(T)
    disp = _sc_scatter(xs, idx)
    eo = _ffn(disp, wi, wo)
    return _sc_gather(eo, idx)


# speedup vs baseline: 2.4136x; 1.0044x over previous
import functools

import jax
import jax.numpy as jnp
from jax import lax
from jax.experimental import pallas as pl
from jax.experimental.pallas import tpu as pltpu
import jax.experimental.pallas.tpu_sc as plsc

T = 4096
D = 1024
D2 = D // 2
F = 4096
E = 8
CAP = 640
SLOT = 648
NROWS = E * SLOT
FB = 2048
NF = F // FB
CHUNK = 1024
NCHUNK = T // CHUNK

NC = 2
NS = 16
NW = NC * NS
TPW = T // NW
SUB = 64
NSUB = TPW // SUB


def _router_body(x_ref, wg_ref, xs_ref, idx_ref, carry_ref):
    c = pl.program_id(0)

    @pl.when(c == 0)
    def _():
        carry_ref[...] = jnp.zeros((1, E), dtype=jnp.float32)

    x = x_ref[...]
    wg = wg_ref[...]
    logits = jnp.dot(x.astype(jnp.bfloat16), wg.astype(jnp.bfloat16),
                     preferred_element_type=jnp.float32)
    m = jnp.max(logits, axis=1, keepdims=True)
    p = jnp.exp(logits - m)
    probs = p / jnp.sum(p, axis=1, keepdims=True)
    gate = jnp.max(probs, axis=1, keepdims=True)
    e_iota = lax.broadcasted_iota(jnp.int32, (CHUNK, E), 1)
    is_max = probs >= gate
    eidx = jnp.min(jnp.where(is_max, e_iota, E), axis=1, keepdims=True)
    mask = (e_iota == eidx).astype(jnp.float32)

    ci = lax.broadcasted_iota(jnp.int32, (CHUNK, CHUNK), 0)
    cj = lax.broadcasted_iota(jnp.int32, (CHUNK, CHUNK), 1)
    tri = (ci >= cj).astype(jnp.float32)
    cum = jnp.dot(tri, mask, preferred_element_type=jnp.float32) + carry_ref[...]
    carry_ref[...] = cum[CHUNK - 1:CHUNK, :]

    pos = jnp.sum(cum * mask, axis=1, keepdims=True) - 1.0
    pos_i = pos.astype(jnp.int32)
    keep = (pos_i < CAP)
    slot = jnp.where(keep, pos_i, CAP)
    idx_ref[...] = eidx * SLOT + slot
    gk = jnp.where(keep, gate, 0.0)
    xs_r = (x * gk).astype(jnp.bfloat16).astype(jnp.float32)
    b = lax.bitcast_convert_type(xs_r, jnp.uint32)
    packed = (b[:, :D2] >> 16) | (b[:, D2:] & jnp.uint32(0xFFFF0000))
    xs_ref[...] = packed


def _router(x, wg):
    return pl.pallas_call(
        _router_body,
        grid=(NCHUNK,),
        in_specs=[
            pl.BlockSpec((CHUNK, D), lambda c: (c, 0)),
            pl.BlockSpec((D, E), lambda c: (0, 0)),
        ],
        out_specs=(pl.BlockSpec((CHUNK, D2), lambda c: (c, 0)),
                   pl.BlockSpec((CHUNK, 1), lambda c: (c, 0))),
        out_shape=(jax.ShapeDtypeStruct((T, D2), jnp.uint32),
                   jax.ShapeDtypeStruct((T, 1), jnp.int32)),
        scratch_shapes=[pltpu.VMEM((1, E), jnp.float32)],
    )(x, wg)


def _ffn_body(disp_ref, wi_ref, wo_ref, eo_ref):
    fb = pl.program_id(1)
    u = disp_ref[...]
    x_lo = lax.bitcast_convert_type(u << 16, jnp.float32)
    x_hi = lax.bitcast_convert_type(u & jnp.uint32(0xFFFF0000),
                                    jnp.float32)
    wi = wi_ref[0]
    wo = wo_ref[0]
    acc = jnp.dot(x_lo, wi[:D2], preferred_element_type=jnp.float32,
                  precision=lax.Precision.DEFAULT)
    acc += jnp.dot(x_hi, wi[D2:], preferred_element_type=jnp.float32,
                   precision=lax.Precision.DEFAULT)
    h = jnp.maximum(acc, 0.0)
    contrib = jnp.dot(h, wo, preferred_element_type=jnp.float32,
                      precision=lax.Precision.DEFAULT)

    @pl.when(fb == 0)
    def _():
        eo_ref[...] = contrib

    @pl.when(fb != 0)
    def _():
        eo_ref[...] += contrib


def _ffn(disp, wi, wo):
    return pl.pallas_call(
        _ffn_body,
        grid=(E, NF),
        in_specs=[
            pl.BlockSpec((SLOT, D2), lambda e, f: (e, 0)),
            pl.BlockSpec((1, D, FB), lambda e, f: (e, 0, f)),
            pl.BlockSpec((1, FB, D), lambda e, f: (e, f, 0)),
        ],
        out_specs=pl.BlockSpec((SLOT, D), lambda e, f: (e, 0)),
        out_shape=jax.ShapeDtypeStruct((NROWS, D), jnp.float32),
    )(disp, wi, wo)


@functools.cache
def _sc_kernels():
    mesh = plsc.VectorSubcoreMesh(core_axis_name="c", subcore_axis_name="s",
                                  num_cores=NC, num_subcores=NS)

    @functools.partial(
        pl.kernel,
        out_type=jax.ShapeDtypeStruct((NROWS, D2), jnp.uint32),
        mesh=mesh,
        scratch_types=[
            pltpu.VMEM((NSUB, SUB), jnp.int32),
            pltpu.VMEM((SUB, D2), jnp.uint32),
            pltpu.SemaphoreType.DMA,
        ])
    def scatter_k(xs_hbm, idx_hbm, disp_hbm, idx_v, rows_v, sem):
        wid = lax.axis_index("s") * NC + lax.axis_index("c")
        base = wid * TPW
        for j in range(NSUB):
            pltpu.sync_copy(idx_hbm.at[pl.ds(base + j * SUB, SUB)], idx_v.at[j])
            pltpu.sync_copy(xs_hbm.at[pl.ds(base + j * SUB, SUB)], rows_v)
            pltpu.async_copy(rows_v, disp_hbm.at[idx_v.at[j]], sem).wait()

    @functools.partial(
        pl.kernel,
        out_type=jax.ShapeDtypeStruct((T, D), jnp.float32),
        mesh=mesh,
        scratch_types=[
            pltpu.VMEM((NSUB, SUB), jnp.int32),
            pltpu.VMEM((SUB, D), jnp.float32),
            pltpu.SemaphoreType.DMA,
        ])
    def gather_k(eo_hbm, idx_hbm, out_hbm, idx_v, rows_v, sem):
        wid = lax.axis_index("s") * NC + lax.axis_index("c")
        base = wid * TPW
        for j in range(NSUB):
            pltpu.sync_copy(idx_hbm.at[pl.ds(base + j * SUB, SUB)], idx_v.at[j])
            pltpu.async_copy(eo_hbm.at[idx_v.at[j]], rows_v, sem).wait()
            pltpu.sync_copy(rows_v, out_hbm.at[pl.ds(base + j * SUB, SUB)])

    return scatter_k, gather_k


def _sc_scatter(xs, idx):
    return _sc_kernels()[0](xs, idx)


def _sc_gather(eo, idx):
    return _sc_kernels()[1](eo, idx)


def kernel(x, Wg, wi, wo):
    xs, idx2d = _router(x, Wg)
    idx = idx2d.reshape(T)
    disp = _sc_scatter(xs, idx)
    eo = _ffn(disp, wi, wo)
    return _sc_gather(eo, idx)
